# baseline (speedup 1.0000x reference)
import jax
import jax.numpy as jnp
from jax.experimental import pallas as pl
from jax.experimental.pallas import tpu as pltpu

_UNROLL = 64


def _hybrid_kernel(wid_ref, pid_ref, wtab_ref, ptab_ref, out_ref, tile_ref):
    tm, dim = out_ref.shape
    n_chunks = dim // 128
    num_pos = ptab_ref.shape[0]

    pids = pid_ref[...]
    iota = jax.lax.broadcasted_iota(jnp.int32, (tm, num_pos), 1)
    oh = (pids == iota).astype(jnp.float32).astype(jnp.bfloat16)
    out_ref[...] = jnp.dot(oh, ptab_ref[...],
                           preferred_element_type=jnp.float32)

    S = tm + 8

    def group(g, carry):
        base = g * _UNROLL
        for u in range(_UNROLL):
            t = base + u
            wi = wid_ref[0, 0, t]
            tile_ref[pl.Slice(t, n_chunks, S), :] = wtab_ref[wi]
        return carry

    jax.lax.fori_loop(0, tm // _UNROLL, group, 0)

    for j in range(n_chunks):
        sl = slice(j * 128, (j + 1) * 128)
        out_ref[:, sl] = out_ref[:, sl] + tile_ref[pl.ds(j * S, tm), :]


def _word_only_kernel(wid_ref, wtab_ref, out_ref):
    tm = out_ref.shape[0]

    def chunk(c, carry):
        base = c * _UNROLL
        for u in range(_UNROLL):
            t = base + u
            wi = wid_ref[0, 0, t]
            out_ref[t, 0] = wtab_ref[wi, 0]
        return carry

    jax.lax.fori_loop(0, tm // _UNROLL, chunk, 0)


def _round_up(x: int, m: int) -> int:
    return ((x + m - 1) // m) * m


def _word_only(word_table, flat_w, n, orig_shape, block_tm):
    vocab, dim = word_table.shape
    tm = max(_UNROLL, min(block_tm, _round_up(n, _UNROLL)))
    n_pad = _round_up(n, tm)
    pad = n_pad - n
    n_blocks = n_pad // tm
    w_ids = jnp.pad(flat_w, (0, pad)).reshape(n_blocks, 1, tm)
    wtab3 = word_table.reshape(vocab, 1, dim)
    out = pl.pallas_call(
        _word_only_kernel,
        out_shape=jax.ShapeDtypeStruct((n_pad, 1, dim), word_table.dtype),
        grid=(n_blocks,),
        in_specs=[
            pl.BlockSpec((1, 1, tm), lambda i: (i, 0, 0),
                         memory_space=pltpu.SMEM),
            pl.BlockSpec((vocab, 1, dim), lambda i: (0, 0, 0)),
        ],
        out_specs=pl.BlockSpec((tm, 1, dim), lambda i: (i, 0, 0)),
        compiler_params=pltpu.CompilerParams(
            dimension_semantics=("arbitrary",),
            vmem_limit_bytes=60 * 1024 * 1024,
        ),
    )(w_ids, wtab3)
    return out[:n, 0].reshape(orig_shape + (dim,))


def seq_gnn_node_embedding_fast(word_table, pos_table, input_ids,
                                position_ids=None, *, add_position=True,
                                block_tm=2048):
    vocab, dim = word_table.shape
    orig_shape = input_ids.shape

    flat_w = input_ids.reshape(-1).astype(jnp.int32)
    n = flat_w.shape[0]
    if n == 0:
        return jnp.zeros(orig_shape + (dim,), dtype=word_table.dtype)

    use_pos = add_position and (position_ids is not None)
    if not use_pos or dim % 128 != 0 or dim > 1024:
        return _word_only(word_table, flat_w, n, orig_shape, block_tm)

    max_pos = pos_table.shape[0]
    tm = max(_UNROLL, min(block_tm, _round_up(n, _UNROLL)))
    n_pad = _round_up(n, tm)
    pad = n_pad - n
    n_blocks = n_pad // tm

    w_ids = jnp.pad(flat_w, (0, pad)).reshape(n_blocks, 1, tm)
    flat_p = jnp.minimum(position_ids.reshape(-1).astype(jnp.int32),
                         max_pos - 1)
    p_ids = jnp.pad(flat_p, (0, pad)).reshape(n_pad, 1)

    n_chunks = dim // 128
    wtab_w = word_table.reshape(vocab, n_chunks, 128)
    ptab_bf = pos_table.astype(jnp.bfloat16)

    out = pl.pallas_call(
        _hybrid_kernel,
        out_shape=jax.ShapeDtypeStruct((n_pad, dim), jnp.float32),
        grid=(n_blocks,),
        in_specs=[
            pl.BlockSpec((1, 1, tm), lambda i: (i, 0, 0),
                         memory_space=pltpu.SMEM),
            pl.BlockSpec((tm, 1), lambda i: (i, 0)),
            pl.BlockSpec((vocab, n_chunks, 128),
                         lambda i: (0, 0, 0)),
            pl.BlockSpec((max_pos, dim), lambda i: (0, 0)),
        ],
        out_specs=pl.BlockSpec((tm, dim), lambda i: (i, 0)),
        scratch_shapes=[pltpu.VMEM((n_chunks * (tm + 8), 128), jnp.float32)],
        compiler_params=pltpu.CompilerParams(
            dimension_semantics=("arbitrary",),
            vmem_limit_bytes=60 * 1024 * 1024,
        ),
    )(w_ids, p_ids, wtab_w, ptab_bf)

    return out[:n].reshape(orig_shape + (dim,))


def kernel(word_table, pos_table, input_ids, position_ids):
    return seq_gnn_node_embedding_fast(word_table, pos_table, input_ids,
                                       position_ids)


# speedup vs baseline: 1.0433x; 1.0433x over previous
import jax
import jax.numpy as jnp
from jax.experimental import pallas as pl
from jax.experimental.pallas import tpu as pltpu

_UNROLL = 32


def _hybrid_kernel(wid_ref, pid_ref, wtab_ref, ptab_ref, out_ref, tile_ref):
    tm, dim = out_ref.shape
    n_chunks = dim // 128
    num_pos = ptab_ref.shape[0]

    pids = pid_ref[...]
    iota = jax.lax.broadcasted_iota(jnp.int32, (tm, num_pos), 1)
    oh = (pids == iota).astype(jnp.float32).astype(jnp.bfloat16)
    out_ref[...] = jnp.dot(oh, ptab_ref[...],
                           preferred_element_type=jnp.float32)

    S = tm + 8

    def group(g, carry):
        base = g * _UNROLL
        for u in range(_UNROLL):
            t = base + u
            wi = wid_ref[0, 0, t]
            tile_ref[pl.Slice(t, n_chunks, S), :] = wtab_ref[wi][:n_chunks]
        return carry

    jax.lax.fori_loop(0, tm // _UNROLL, group, 0)

    for j in range(n_chunks):
        sl = slice(j * 128, (j + 1) * 128)
        out_ref[:, sl] = out_ref[:, sl] + tile_ref[pl.ds(j * S, tm), :]


def _word_only_kernel(wid_ref, wtab_ref, out_ref):
    tm = out_ref.shape[0]

    def chunk(c, carry):
        base = c * _UNROLL
        for u in range(_UNROLL):
            t = base + u
            wi = wid_ref[0, 0, t]
            out_ref[t, 0] = wtab_ref[wi, 0]
        return carry

    jax.lax.fori_loop(0, tm // _UNROLL, chunk, 0)


def _round_up(x: int, m: int) -> int:
    return ((x + m - 1) // m) * m


def _word_only(word_table, flat_w, n, orig_shape, block_tm):
    vocab, dim = word_table.shape
    tm = max(_UNROLL, min(block_tm, _round_up(n, _UNROLL)))
    n_pad = _round_up(n, tm)
    pad = n_pad - n
    n_blocks = n_pad // tm
    w_ids = jnp.pad(flat_w, (0, pad)).reshape(n_blocks, 1, tm)
    wtab3 = word_table.reshape(vocab, 1, dim)
    out = pl.pallas_call(
        _word_only_kernel,
        out_shape=jax.ShapeDtypeStruct((n_pad, 1, dim), word_table.dtype),
        grid=(n_blocks,),
        in_specs=[
            pl.BlockSpec((1, 1, tm), lambda i: (i, 0, 0),
                         memory_space=pltpu.SMEM),
            pl.BlockSpec((vocab, 1, dim), lambda i: (0, 0, 0)),
        ],
        out_specs=pl.BlockSpec((tm, 1, dim), lambda i: (i, 0, 0)),
        compiler_params=pltpu.CompilerParams(
            dimension_semantics=("arbitrary",),
            vmem_limit_bytes=60 * 1024 * 1024,
        ),
    )(w_ids, wtab3)
    return out[:n, 0].reshape(orig_shape + (dim,))


def seq_gnn_node_embedding_fast(word_table, pos_table, input_ids,
                                position_ids=None, *, add_position=True,
                                block_tm=2048):
    vocab, dim = word_table.shape
    orig_shape = input_ids.shape

    flat_w = input_ids.reshape(-1).astype(jnp.int32)
    n = flat_w.shape[0]
    if n == 0:
        return jnp.zeros(orig_shape + (dim,), dtype=word_table.dtype)

    use_pos = add_position and (position_ids is not None)
    if not use_pos or dim % 128 != 0 or dim > 1024:
        return _word_only(word_table, flat_w, n, orig_shape, block_tm)

    max_pos = pos_table.shape[0]
    tm = max(_UNROLL, min(block_tm, _round_up(n, _UNROLL)))
    n_pad = _round_up(n, tm)
    pad = n_pad - n
    n_blocks = n_pad // tm

    w_ids = jnp.pad(flat_w, (0, pad)).reshape(n_blocks, 1, tm)
    flat_p = jnp.minimum(position_ids.reshape(-1).astype(jnp.int32),
                         max_pos - 1)
    p_ids = jnp.pad(flat_p, (0, pad)).reshape(n_pad, 1)

    n_chunks = dim // 128
    wtab_w = jnp.pad(word_table, ((0, 0), (0, 1024 - dim))).reshape(
        vocab, 8, 128)
    ptab_bf = pos_table.astype(jnp.bfloat16)

    out = pl.pallas_call(
        _hybrid_kernel,
        out_shape=jax.ShapeDtypeStruct((n_pad, dim), jnp.float32),
        grid=(n_blocks,),
        in_specs=[
            pl.BlockSpec((1, 1, tm), lambda i: (i, 0, 0),
                         memory_space=pltpu.SMEM),
            pl.BlockSpec((tm, 1), lambda i: (i, 0)),
            pl.BlockSpec((vocab, 8, 128),
                         lambda i: (0, 0, 0)),
            pl.BlockSpec((max_pos, dim), lambda i: (0, 0)),
        ],
        out_specs=pl.BlockSpec((tm, dim), lambda i: (i, 0)),
        scratch_shapes=[pltpu.VMEM((n_chunks * (tm + 8), 128), jnp.float32)],
        compiler_params=pltpu.CompilerParams(
            dimension_semantics=("arbitrary",),
            vmem_limit_bytes=60 * 1024 * 1024,
        ),
    )(w_ids, p_ids, wtab_w, ptab_bf)

    return out[:n].reshape(orig_shape + (dim,))


def kernel(word_table, pos_table, input_ids, position_ids):
    return seq_gnn_node_embedding_fast(word_table, pos_table, input_ids,
                                       position_ids)


# speedup vs baseline: 1.3231x; 1.2681x over previous
import jax
import jax.numpy as jnp
from jax.experimental import pallas as pl
from jax.experimental.pallas import tpu as pltpu

_UNROLL = 32


_SUB = 256


def _hybrid_kernel(wid_ref, pid_ref, wtab_ref, ptab_ref, out_ref, tile_ref):
    tm, dim = out_ref.shape
    n_chunks = dim // 128
    num_pos = ptab_ref.shape[0]
    ptab = ptab_ref[...]

    S = tm + 8
    for b in range(tm // _SUB):
        rows = pl.ds(b * _SUB, _SUB)

        pids = pid_ref[rows, :]
        iota = jax.lax.broadcasted_iota(jnp.int32, (_SUB, num_pos), 1)
        oh = (pids == iota).astype(jnp.float32).astype(jnp.bfloat16)
        out_ref[rows, :] = jnp.dot(oh, ptab,
                                   preferred_element_type=jnp.float32)

        for u in range(_SUB):
            t = b * _SUB + u
            wi = wid_ref[0, 0, t]
            tile_ref[pl.Slice(t, n_chunks, S), :] = wtab_ref[wi][:n_chunks]

        for j in range(n_chunks):
            sl = slice(j * 128, (j + 1) * 128)
            out_ref[rows, sl] = out_ref[rows, sl] + \
                tile_ref[pl.ds(j * S + b * _SUB, _SUB), :]


def _word_only_kernel(wid_ref, wtab_ref, out_ref):
    tm = out_ref.shape[0]

    def chunk(c, carry):
        base = c * _UNROLL
        for u in range(_UNROLL):
            t = base + u
            wi = wid_ref[0, 0, t]
            out_ref[t, 0] = wtab_ref[wi, 0]
        return carry

    jax.lax.fori_loop(0, tm // _UNROLL, chunk, 0)


def _round_up(x: int, m: int) -> int:
    return ((x + m - 1) // m) * m


def _word_only(word_table, flat_w, n, orig_shape, block_tm):
    vocab, dim = word_table.shape
    tm = max(_UNROLL, min(block_tm, _round_up(n, _UNROLL)))
    n_pad = _round_up(n, tm)
    pad = n_pad - n
    n_blocks = n_pad // tm
    w_ids = jnp.pad(flat_w, (0, pad)).reshape(n_blocks, 1, tm)
    wtab3 = word_table.reshape(vocab, 1, dim)
    out = pl.pallas_call(
        _word_only_kernel,
        out_shape=jax.ShapeDtypeStruct((n_pad, 1, dim), word_table.dtype),
        grid=(n_blocks,),
        in_specs=[
            pl.BlockSpec((1, 1, tm), lambda i: (i, 0, 0),
                         memory_space=pltpu.SMEM),
            pl.BlockSpec((vocab, 1, dim), lambda i: (0, 0, 0)),
        ],
        out_specs=pl.BlockSpec((tm, 1, dim), lambda i: (i, 0, 0)),
        compiler_params=pltpu.CompilerParams(
            dimension_semantics=("arbitrary",),
            vmem_limit_bytes=60 * 1024 * 1024,
        ),
    )(w_ids, wtab3)
    return out[:n, 0].reshape(orig_shape + (dim,))


def seq_gnn_node_embedding_fast(word_table, pos_table, input_ids,
                                position_ids=None, *, add_position=True,
                                block_tm=2048):
    vocab, dim = word_table.shape
    orig_shape = input_ids.shape

    flat_w = input_ids.reshape(-1).astype(jnp.int32)
    n = flat_w.shape[0]
    if n == 0:
        return jnp.zeros(orig_shape + (dim,), dtype=word_table.dtype)

    use_pos = add_position and (position_ids is not None)
    if not use_pos or dim % 128 != 0 or dim > 1024:
        return _word_only(word_table, flat_w, n, orig_shape, block_tm)

    max_pos = pos_table.shape[0]
    tm = max(_UNROLL, min(block_tm, _round_up(n, _UNROLL)))
    n_pad = _round_up(n, tm)
    pad = n_pad - n
    n_blocks = n_pad // tm

    w_ids = jnp.pad(flat_w, (0, pad)).reshape(n_blocks, 1, tm)
    flat_p = jnp.minimum(position_ids.reshape(-1).astype(jnp.int32),
                         max_pos - 1)
    p_ids = jnp.pad(flat_p, (0, pad)).reshape(n_pad, 1)

    n_chunks = dim // 128
    wtab_w = jnp.pad(word_table, ((0, 0), (0, 1024 - dim))).reshape(
        vocab, 8, 128)
    ptab_bf = pos_table.astype(jnp.bfloat16)

    out = pl.pallas_call(
        _hybrid_kernel,
        out_shape=jax.ShapeDtypeStruct((n_pad, dim), jnp.float32),
        grid=(n_blocks,),
        in_specs=[
            pl.BlockSpec((1, 1, tm), lambda i: (i, 0, 0),
                         memory_space=pltpu.SMEM),
            pl.BlockSpec((tm, 1), lambda i: (i, 0)),
            pl.BlockSpec((vocab, 8, 128),
                         lambda i: (0, 0, 0)),
            pl.BlockSpec((max_pos, dim), lambda i: (0, 0)),
        ],
        out_specs=pl.BlockSpec((tm, dim), lambda i: (i, 0)),
        scratch_shapes=[pltpu.VMEM((n_chunks * (tm + 8), 128), jnp.float32)],
        compiler_params=pltpu.CompilerParams(
            dimension_semantics=("arbitrary",),
            vmem_limit_bytes=60 * 1024 * 1024,
        ),
    )(w_ids, p_ids, wtab_w, ptab_bf)

    return out[:n].reshape(orig_shape + (dim,))


def kernel(word_table, pos_table, input_ids, position_ids):
    return seq_gnn_node_embedding_fast(word_table, pos_table, input_ids,
                                       position_ids)


# speedup vs baseline: 1.3392x; 1.0122x over previous
import jax
import jax.numpy as jnp
from jax.experimental import pallas as pl
from jax.experimental.pallas import tpu as pltpu

_UNROLL = 32


_SUB = 512


def _hybrid_kernel(wid_ref, pid_ref, wtab_ref, ptab_ref, out_ref, tile_ref):
    tm, dim = out_ref.shape
    n_chunks = dim // 128
    num_pos = ptab_ref.shape[0]
    ptab = ptab_ref[...].astype(jnp.bfloat16)

    S = tm + 8
    for b in range(tm // _SUB):
        rows = pl.ds(b * _SUB, _SUB)

        pids = jnp.minimum(pid_ref[rows, :], num_pos - 1)
        iota = jax.lax.broadcasted_iota(jnp.int32, (_SUB, num_pos), 1)
        oh = (pids == iota).astype(jnp.float32).astype(jnp.bfloat16)
        out_ref[rows, :] = jnp.dot(oh, ptab,
                                   preferred_element_type=jnp.float32)

        for u in range(_SUB):
            t = b * _SUB + u
            wi = wid_ref[0, 0, t]
            tile_ref[pl.Slice(t, n_chunks, S), :] = wtab_ref[wi][:n_chunks]

        for j in range(n_chunks):
            sl = slice(j * 128, (j + 1) * 128)
            out_ref[rows, sl] = out_ref[rows, sl] + \
                tile_ref[pl.ds(j * S + b * _SUB, _SUB), :]


def _word_only_kernel(wid_ref, wtab_ref, out_ref):
    tm = out_ref.shape[0]

    def chunk(c, carry):
        base = c * _UNROLL
        for u in range(_UNROLL):
            t = base + u
            wi = wid_ref[0, 0, t]
            out_ref[t, 0] = wtab_ref[wi, 0]
        return carry

    jax.lax.fori_loop(0, tm // _UNROLL, chunk, 0)


def _round_up(x: int, m: int) -> int:
    return ((x + m - 1) // m) * m


def _word_only(word_table, flat_w, n, orig_shape, block_tm):
    vocab, dim = word_table.shape
    tm = max(_UNROLL, min(block_tm, _round_up(n, _UNROLL)))
    n_pad = _round_up(n, tm)
    pad = n_pad - n
    n_blocks = n_pad // tm
    w_ids = jnp.pad(flat_w, (0, pad)).reshape(n_blocks, 1, tm)
    wtab3 = word_table.reshape(vocab, 1, dim)
    out = pl.pallas_call(
        _word_only_kernel,
        out_shape=jax.ShapeDtypeStruct((n_pad, 1, dim), word_table.dtype),
        grid=(n_blocks,),
        in_specs=[
            pl.BlockSpec((1, 1, tm), lambda i: (i, 0, 0),
                         memory_space=pltpu.SMEM),
            pl.BlockSpec((vocab, 1, dim), lambda i: (0, 0, 0)),
        ],
        out_specs=pl.BlockSpec((tm, 1, dim), lambda i: (i, 0, 0)),
        compiler_params=pltpu.CompilerParams(
            dimension_semantics=("arbitrary",),
            vmem_limit_bytes=60 * 1024 * 1024,
        ),
    )(w_ids, wtab3)
    return out[:n, 0].reshape(orig_shape + (dim,))


def seq_gnn_node_embedding_fast(word_table, pos_table, input_ids,
                                position_ids=None, *, add_position=True,
                                block_tm=2048):
    vocab, dim = word_table.shape
    orig_shape = input_ids.shape

    flat_w = input_ids.reshape(-1).astype(jnp.int32)
    n = flat_w.shape[0]
    if n == 0:
        return jnp.zeros(orig_shape + (dim,), dtype=word_table.dtype)

    use_pos = add_position and (position_ids is not None)
    if not use_pos or dim % 128 != 0 or dim > 1024:
        return _word_only(word_table, flat_w, n, orig_shape, block_tm)

    max_pos = pos_table.shape[0]
    tm = max(_UNROLL, min(block_tm, _round_up(n, _UNROLL)))
    n_pad = _round_up(n, tm)
    pad = n_pad - n
    n_blocks = n_pad // tm

    w_ids = jnp.pad(flat_w, (0, pad)).reshape(n_blocks, 1, tm)
    flat_p = position_ids.reshape(-1).astype(jnp.int32)
    p_ids = jnp.pad(flat_p, (0, pad)).reshape(n_pad, 1)

    n_chunks = dim // 128
    wtab_w = jnp.pad(word_table, ((0, 0), (0, 1024 - dim))).reshape(
        vocab, 8, 128)

    out = pl.pallas_call(
        _hybrid_kernel,
        out_shape=jax.ShapeDtypeStruct((n_pad, dim), jnp.float32),
        grid=(n_blocks,),
        in_specs=[
            pl.BlockSpec((1, 1, tm), lambda i: (i, 0, 0),
                         memory_space=pltpu.SMEM),
            pl.BlockSpec((tm, 1), lambda i: (i, 0)),
            pl.BlockSpec((vocab, 8, 128),
                         lambda i: (0, 0, 0)),
            pl.BlockSpec((max_pos, dim), lambda i: (0, 0)),
        ],
        out_specs=pl.BlockSpec((tm, dim), lambda i: (i, 0)),
        scratch_shapes=[pltpu.VMEM((n_chunks * (tm + 8), 128), jnp.float32)],
        compiler_params=pltpu.CompilerParams(
            dimension_semantics=("arbitrary",),
            vmem_limit_bytes=60 * 1024 * 1024,
        ),
    )(w_ids, p_ids, wtab_w, pos_table)

    return out[:n].reshape(orig_shape + (dim,))


def kernel(word_table, pos_table, input_ids, position_ids):
    return seq_gnn_node_embedding_fast(word_table, pos_table, input_ids,
                                       position_ids)


# speedup vs baseline: 1.5563x; 1.1621x over previous
import jax
import jax.numpy as jnp
from jax.experimental import pallas as pl
from jax.experimental.pallas import tpu as pltpu

_UNROLL = 32


_SUB = 512


def _hybrid_kernel(wid_ref, pid_ref, wtab_ref, ptab_ref, out_ref,
                   tile_ref, wrap_ref, stage_ref, dma_sem):
    tm, dim = out_ref.shape
    n_chunks = dim // 128
    num_pos = ptab_ref.shape[0]
    vocab = wtab_ref.shape[0]
    ptab = ptab_ref[...].astype(jnp.bfloat16)

    @pl.when(pl.program_id(0) == 0)
    def _build_wrap():
        nrows = 512
        def body(c, carry):
            pltpu.make_async_copy(
                wtab_ref.at[pl.ds(c * nrows, nrows), :],
                stage_ref, dma_sem).start()
            pltpu.make_async_copy(
                wtab_ref.at[pl.ds(c * nrows, nrows), :],
                stage_ref, dma_sem).wait()
            for j in range(n_chunks):
                wrap_ref[pl.Slice(8 * nrows * c + j, nrows, 8), :] = \
                    stage_ref[:, j * 128:(j + 1) * 128]
            return carry
        jax.lax.fori_loop(0, vocab // nrows, body, 0)

    S = tm + 8
    for b in range(tm // _SUB):
        rows = pl.ds(b * _SUB, _SUB)

        pids = jnp.minimum(pid_ref[rows, :], num_pos - 1)
        iota = jax.lax.broadcasted_iota(jnp.int32, (_SUB, num_pos), 1)
        oh = (pids == iota).astype(jnp.float32).astype(jnp.bfloat16)
        out_ref[rows, :] = jnp.dot(oh, ptab,
                                   preferred_element_type=jnp.float32)

        for u in range(_SUB):
            t = b * _SUB + u
            wi8 = pl.multiple_of(wid_ref[0, 0, t] * 8, 8)
            tile_ref[pl.Slice(t, n_chunks, S), :] = \
                wrap_ref[pl.ds(wi8, 8), :][:n_chunks]

        for j in range(n_chunks):
            sl = slice(j * 128, (j + 1) * 128)
            out_ref[rows, sl] = out_ref[rows, sl] + \
                tile_ref[pl.ds(j * S + b * _SUB, _SUB), :]


def _word_only_kernel(wid_ref, wtab_ref, out_ref):
    tm = out_ref.shape[0]

    def chunk(c, carry):
        base = c * _UNROLL
        for u in range(_UNROLL):
            t = base + u
            wi = wid_ref[0, 0, t]
            out_ref[t, 0] = wtab_ref[wi, 0]
        return carry

    jax.lax.fori_loop(0, tm // _UNROLL, chunk, 0)


def _round_up(x: int, m: int) -> int:
    return ((x + m - 1) // m) * m


def _word_only(word_table, flat_w, n, orig_shape, block_tm):
    vocab, dim = word_table.shape
    tm = max(_UNROLL, min(block_tm, _round_up(n, _UNROLL)))
    n_pad = _round_up(n, tm)
    pad = n_pad - n
    n_blocks = n_pad // tm
    w_ids = jnp.pad(flat_w, (0, pad)).reshape(n_blocks, 1, tm)
    wtab3 = word_table.reshape(vocab, 1, dim)
    out = pl.pallas_call(
        _word_only_kernel,
        out_shape=jax.ShapeDtypeStruct((n_pad, 1, dim), word_table.dtype),
        grid=(n_blocks,),
        in_specs=[
            pl.BlockSpec((1, 1, tm), lambda i: (i, 0, 0),
                         memory_space=pltpu.SMEM),
            pl.BlockSpec((vocab, 1, dim), lambda i: (0, 0, 0)),
        ],
        out_specs=pl.BlockSpec((tm, 1, dim), lambda i: (i, 0, 0)),
        compiler_params=pltpu.CompilerParams(
            dimension_semantics=("arbitrary",),
            vmem_limit_bytes=60 * 1024 * 1024,
        ),
    )(w_ids, wtab3)
    return out[:n, 0].reshape(orig_shape + (dim,))


def seq_gnn_node_embedding_fast(word_table, pos_table, input_ids,
                                position_ids=None, *, add_position=True,
                                block_tm=2048):
    vocab, dim = word_table.shape
    orig_shape = input_ids.shape

    flat_w = input_ids.reshape(-1).astype(jnp.int32)
    n = flat_w.shape[0]
    if n == 0:
        return jnp.zeros(orig_shape + (dim,), dtype=word_table.dtype)

    use_pos = add_position and (position_ids is not None)
    if not use_pos or dim % 128 != 0 or dim > 1024:
        return _word_only(word_table, flat_w, n, orig_shape, block_tm)

    max_pos = pos_table.shape[0]
    tm = max(_UNROLL, min(block_tm, _round_up(n, _UNROLL)))
    n_pad = _round_up(n, tm)
    pad = n_pad - n
    n_blocks = n_pad // tm

    w_ids = jnp.pad(flat_w, (0, pad)).reshape(n_blocks, 1, tm)
    flat_p = position_ids.reshape(-1).astype(jnp.int32)
    p_ids = jnp.pad(flat_p, (0, pad)).reshape(n_pad, 1)

    n_chunks = dim // 128

    out = pl.pallas_call(
        _hybrid_kernel,
        out_shape=jax.ShapeDtypeStruct((n_pad, dim), jnp.float32),
        grid=(n_blocks,),
        in_specs=[
            pl.BlockSpec((1, 1, tm), lambda i: (i, 0, 0),
                         memory_space=pltpu.SMEM),
            pl.BlockSpec((tm, 1), lambda i: (i, 0)),
            pl.BlockSpec(memory_space=pl.ANY),
            pl.BlockSpec((max_pos, dim), lambda i: (0, 0)),
        ],
        out_specs=pl.BlockSpec((tm, dim), lambda i: (i, 0)),
        scratch_shapes=[
            pltpu.VMEM((n_chunks * (tm + 8), 128), jnp.float32),
            pltpu.VMEM((vocab * 8, 128), jnp.float32),
            pltpu.VMEM((512, dim), jnp.float32),
            pltpu.SemaphoreType.DMA,
        ],
        compiler_params=pltpu.CompilerParams(
            dimension_semantics=("arbitrary",),
            vmem_limit_bytes=60 * 1024 * 1024,
        ),
    )(w_ids, p_ids, word_table, pos_table)

    return out[:n].reshape(orig_shape + (dim,))


def kernel(word_table, pos_table, input_ids, position_ids):
    return seq_gnn_node_embedding_fast(word_table, pos_table, input_ids,
                                       position_ids)


# speedup vs baseline: 1.5605x; 1.0027x over previous
import jax
import jax.numpy as jnp
from jax.experimental import pallas as pl
from jax.experimental.pallas import tpu as pltpu

_UNROLL = 32


_SUB = 512


def _hybrid_kernel(wid_ref, pid_ref, wtab_ref, ptab_ref, out_ref,
                   tile_ref, wrap_ref, stage_ref, dma_sem):
    tm, dim = out_ref.shape
    n_chunks = dim // 128
    num_pos = ptab_ref.shape[0]
    vocab = wtab_ref.shape[0]
    ptab = ptab_ref[...].astype(jnp.bfloat16)

    @pl.when(pl.program_id(0) == 0)
    def _build_wrap():
        nrows = 512
        def body(c, carry):
            pltpu.make_async_copy(
                wtab_ref.at[pl.ds(c * nrows, nrows), :],
                stage_ref, dma_sem).start()
            pltpu.make_async_copy(
                wtab_ref.at[pl.ds(c * nrows, nrows), :],
                stage_ref, dma_sem).wait()
            for j in range(n_chunks):
                wrap_ref[pl.Slice(8 * nrows * c + j, nrows, 8), :] = \
                    stage_ref[:, j * 128:(j + 1) * 128]
            return carry
        jax.lax.fori_loop(0, vocab // nrows, body, 0)

    S = tm + 8
    for b in range(tm // _SUB):
        rows = pl.ds(b * _SUB, _SUB)

        pids = jnp.minimum(pid_ref[rows, :], num_pos - 1)
        iota = jax.lax.broadcasted_iota(jnp.int32, (_SUB, num_pos), 1)
        oh = (pids == iota).astype(jnp.float32).astype(jnp.bfloat16)
        out_ref[rows, :] = jnp.dot(oh, ptab,
                                   preferred_element_type=jnp.float32)

        for u0 in range(0, _SUB, 16):
            slabs = []
            for u in range(u0, u0 + 16):
                wi8 = pl.multiple_of(wid_ref[0, 0, b * _SUB + u] * 8, 8)
                slabs.append(wrap_ref[pl.ds(wi8, 8), :][:n_chunks])
            for k, slab in enumerate(slabs):
                t = b * _SUB + u0 + k
                tile_ref[pl.Slice(t, n_chunks, S), :] = slab

        for j in range(n_chunks):
            sl = slice(j * 128, (j + 1) * 128)
            out_ref[rows, sl] = out_ref[rows, sl] + \
                tile_ref[pl.ds(j * S + b * _SUB, _SUB), :]


def _word_only_kernel(wid_ref, wtab_ref, out_ref):
    tm = out_ref.shape[0]

    def chunk(c, carry):
        base = c * _UNROLL
        for u in range(_UNROLL):
            t = base + u
            wi = wid_ref[0, 0, t]
            out_ref[t, 0] = wtab_ref[wi, 0]
        return carry

    jax.lax.fori_loop(0, tm // _UNROLL, chunk, 0)


def _round_up(x: int, m: int) -> int:
    return ((x + m - 1) // m) * m


def _word_only(word_table, flat_w, n, orig_shape, block_tm):
    vocab, dim = word_table.shape
    tm = max(_UNROLL, min(block_tm, _round_up(n, _UNROLL)))
    n_pad = _round_up(n, tm)
    pad = n_pad - n
    n_blocks = n_pad // tm
    w_ids = jnp.pad(flat_w, (0, pad)).reshape(n_blocks, 1, tm)
    wtab3 = word_table.reshape(vocab, 1, dim)
    out = pl.pallas_call(
        _word_only_kernel,
        out_shape=jax.ShapeDtypeStruct((n_pad, 1, dim), word_table.dtype),
        grid=(n_blocks,),
        in_specs=[
            pl.BlockSpec((1, 1, tm), lambda i: (i, 0, 0),
                         memory_space=pltpu.SMEM),
            pl.BlockSpec((vocab, 1, dim), lambda i: (0, 0, 0)),
        ],
        out_specs=pl.BlockSpec((tm, 1, dim), lambda i: (i, 0, 0)),
        compiler_params=pltpu.CompilerParams(
            dimension_semantics=("arbitrary",),
            vmem_limit_bytes=60 * 1024 * 1024,
        ),
    )(w_ids, wtab3)
    return out[:n, 0].reshape(orig_shape + (dim,))


def seq_gnn_node_embedding_fast(word_table, pos_table, input_ids,
                                position_ids=None, *, add_position=True,
                                block_tm=2048):
    vocab, dim = word_table.shape
    orig_shape = input_ids.shape

    flat_w = input_ids.reshape(-1).astype(jnp.int32)
    n = flat_w.shape[0]
    if n == 0:
        return jnp.zeros(orig_shape + (dim,), dtype=word_table.dtype)

    use_pos = add_position and (position_ids is not None)
    if not use_pos or dim % 128 != 0 or dim > 1024:
        return _word_only(word_table, flat_w, n, orig_shape, block_tm)

    max_pos = pos_table.shape[0]
    tm = max(_UNROLL, min(block_tm, _round_up(n, _UNROLL)))
    n_pad = _round_up(n, tm)
    pad = n_pad - n
    n_blocks = n_pad // tm

    w_ids = jnp.pad(flat_w, (0, pad)).reshape(n_blocks, 1, tm)
    flat_p = position_ids.reshape(-1).astype(jnp.int32)
    p_ids = jnp.pad(flat_p, (0, pad)).reshape(n_pad, 1)

    n_chunks = dim // 128

    out = pl.pallas_call(
        _hybrid_kernel,
        out_shape=jax.ShapeDtypeStruct((n_pad, dim), jnp.float32),
        grid=(n_blocks,),
        in_specs=[
            pl.BlockSpec((1, 1, tm), lambda i: (i, 0, 0),
                         memory_space=pltpu.SMEM),
            pl.BlockSpec((tm, 1), lambda i: (i, 0)),
            pl.BlockSpec(memory_space=pl.ANY),
            pl.BlockSpec((max_pos, dim), lambda i: (0, 0)),
        ],
        out_specs=pl.BlockSpec((tm, dim), lambda i: (i, 0)),
        scratch_shapes=[
            pltpu.VMEM((n_chunks * (tm + 8), 128), jnp.float32),
            pltpu.VMEM((vocab * 8, 128), jnp.float32),
            pltpu.VMEM((512, dim), jnp.float32),
            pltpu.SemaphoreType.DMA,
        ],
        compiler_params=pltpu.CompilerParams(
            dimension_semantics=("arbitrary",),
            vmem_limit_bytes=60 * 1024 * 1024,
        ),
    )(w_ids, p_ids, word_table, pos_table)

    return out[:n].reshape(orig_shape + (dim,))


def kernel(word_table, pos_table, input_ids, position_ids):
    return seq_gnn_node_embedding_fast(word_table, pos_table, input_ids,
                                       position_ids)


# speedup vs baseline: 1.6897x; 1.0828x over previous
import jax
import jax.numpy as jnp
from jax.experimental import pallas as pl
from jax.experimental.pallas import tpu as pltpu

_UNROLL = 32


_SUB = 512


def _hybrid_kernel(wid_ref, pid_ref, wtab_ref, ptab_ref, out_ref,
                   tile_ref, wrap_ref, stage_ref, dma_sem):
    tm, dim = out_ref.shape
    n_chunks = dim // 128
    num_pos = ptab_ref.shape[0]
    vocab = wtab_ref.shape[0]
    ptab = ptab_ref[...].astype(jnp.bfloat16)

    @pl.when(pl.program_id(0) == 0)
    def _build_wrap():
        nrows = 512
        def body(c, carry):
            pltpu.make_async_copy(
                wtab_ref.at[pl.ds(c * nrows, nrows), :],
                stage_ref, dma_sem).start()
            pltpu.make_async_copy(
                wtab_ref.at[pl.ds(c * nrows, nrows), :],
                stage_ref, dma_sem).wait()
            for j in range(n_chunks):
                wrap_ref[pl.Slice(8 * nrows * c + j, nrows, 8), :] = \
                    stage_ref[:, j * 128:(j + 1) * 128]
            return carry
        jax.lax.fori_loop(0, vocab // nrows, body, 0)

    S = tm + 4
    for b in range(tm // _SUB):
        rows = pl.ds(b * _SUB, _SUB)

        pids = jnp.minimum(pid_ref[rows, :], num_pos - 1)
        iota = jax.lax.broadcasted_iota(jnp.int32, (_SUB, num_pos), 1)
        oh = (pids == iota).astype(jnp.float32).astype(jnp.bfloat16)
        out_ref[rows, :] = jnp.dot(oh, ptab,
                                   preferred_element_type=jnp.float32)

        for u0 in range(0, _SUB, 16):
            slabs = []
            for u in range(u0, u0 + 16):
                wi8 = pl.multiple_of(wid_ref[0, 0, b * _SUB + u] * 8, 8)
                slabs.append(wrap_ref[pl.ds(wi8, 8), :][:n_chunks])
            for k, slab in enumerate(slabs):
                t = b * _SUB + u0 + k
                tile_ref[pl.Slice(t, n_chunks, S), :] = slab

        for j in range(n_chunks):
            sl = slice(j * 128, (j + 1) * 128)
            out_ref[rows, sl] = out_ref[rows, sl] + \
                tile_ref[pl.ds(j * S + b * _SUB, _SUB), :]


def _word_only_kernel(wid_ref, wtab_ref, out_ref):
    tm = out_ref.shape[0]

    def chunk(c, carry):
        base = c * _UNROLL
        for u in range(_UNROLL):
            t = base + u
            wi = wid_ref[0, 0, t]
            out_ref[t, 0] = wtab_ref[wi, 0]
        return carry

    jax.lax.fori_loop(0, tm // _UNROLL, chunk, 0)


def _round_up(x: int, m: int) -> int:
    return ((x + m - 1) // m) * m


def _word_only(word_table, flat_w, n, orig_shape, block_tm):
    vocab, dim = word_table.shape
    tm = max(_UNROLL, min(block_tm, _round_up(n, _UNROLL)))
    n_pad = _round_up(n, tm)
    pad = n_pad - n
    n_blocks = n_pad // tm
    w_ids = jnp.pad(flat_w, (0, pad)).reshape(n_blocks, 1, tm)
    wtab3 = word_table.reshape(vocab, 1, dim)
    out = pl.pallas_call(
        _word_only_kernel,
        out_shape=jax.ShapeDtypeStruct((n_pad, 1, dim), word_table.dtype),
        grid=(n_blocks,),
        in_specs=[
            pl.BlockSpec((1, 1, tm), lambda i: (i, 0, 0),
                         memory_space=pltpu.SMEM),
            pl.BlockSpec((vocab, 1, dim), lambda i: (0, 0, 0)),
        ],
        out_specs=pl.BlockSpec((tm, 1, dim), lambda i: (i, 0, 0)),
        compiler_params=pltpu.CompilerParams(
            dimension_semantics=("arbitrary",),
            vmem_limit_bytes=60 * 1024 * 1024,
        ),
    )(w_ids, wtab3)
    return out[:n, 0].reshape(orig_shape + (dim,))


def seq_gnn_node_embedding_fast(word_table, pos_table, input_ids,
                                position_ids=None, *, add_position=True,
                                block_tm=2048):
    vocab, dim = word_table.shape
    orig_shape = input_ids.shape

    flat_w = input_ids.reshape(-1).astype(jnp.int32)
    n = flat_w.shape[0]
    if n == 0:
        return jnp.zeros(orig_shape + (dim,), dtype=word_table.dtype)

    use_pos = add_position and (position_ids is not None)
    if not use_pos or dim % 128 != 0 or dim > 1024:
        return _word_only(word_table, flat_w, n, orig_shape, block_tm)

    max_pos = pos_table.shape[0]
    tm = max(_UNROLL, min(block_tm, _round_up(n, _UNROLL)))
    n_pad = _round_up(n, tm)
    pad = n_pad - n
    n_blocks = n_pad // tm

    w_ids = jnp.pad(flat_w, (0, pad)).reshape(n_blocks, 1, tm)
    flat_p = position_ids.reshape(-1).astype(jnp.int32)
    p_ids = jnp.pad(flat_p, (0, pad)).reshape(n_pad, 1)

    n_chunks = dim // 128

    out = pl.pallas_call(
        _hybrid_kernel,
        out_shape=jax.ShapeDtypeStruct((n_pad, dim), jnp.float32),
        grid=(n_blocks,),
        in_specs=[
            pl.BlockSpec((1, 1, tm), lambda i: (i, 0, 0),
                         memory_space=pltpu.SMEM),
            pl.BlockSpec((tm, 1), lambda i: (i, 0)),
            pl.BlockSpec(memory_space=pl.ANY),
            pl.BlockSpec((max_pos, dim), lambda i: (0, 0)),
        ],
        out_specs=pl.BlockSpec((tm, dim), lambda i: (i, 0)),
        scratch_shapes=[
            pltpu.VMEM((n_chunks * (tm + 4) + 8, 128), jnp.float32),
            pltpu.VMEM((vocab * 8, 128), jnp.float32),
            pltpu.VMEM((512, dim), jnp.float32),
            pltpu.SemaphoreType.DMA,
        ],
        compiler_params=pltpu.CompilerParams(
            dimension_semantics=("arbitrary",),
            vmem_limit_bytes=60 * 1024 * 1024,
        ),
    )(w_ids, p_ids, word_table, pos_table)

    return out[:n].reshape(orig_shape + (dim,))


def kernel(word_table, pos_table, input_ids, position_ids):
    return seq_gnn_node_embedding_fast(word_table, pos_table, input_ids,
                                       position_ids)
